# baseline (speedup 1.0000x reference)
import functools
import math

import jax
import jax.numpy as jnp
from jax import lax
from jax.experimental import pallas as pl
from jax.experimental.pallas import tpu as pltpu
from jax.experimental.pallas import tpu_sc as plsc

D_MODEL = 128
LANES = 16
NUM_CORES = 2
NUM_SUBCORES = 16
CHUNK = 64
FAST_CHUNKS = 9
SLOW_CHUNKS = 7
SCALE = math.sqrt(float(D_MODEL))


@functools.lru_cache(maxsize=None)
def _build(b0: int, b1: int):
    batch = b0 * b1
    nchunks_total = batch // CHUNK
    assert nchunks_total == NUM_SUBCORES * (FAST_CHUNKS + SLOW_CHUNKS)
    assert b1 % CHUNK == 0

    mesh = plsc.VectorSubcoreMesh(core_axis_name="c", subcore_axis_name="s",
                                  num_cores=NUM_CORES,
                                  num_subcores=NUM_SUBCORES)

    @functools.partial(
        pl.kernel,
        out_type=jax.ShapeDtypeStruct((batch, D_MODEL), jnp.float32),
        mesh=mesh,
        scratch_types=[
            pltpu.VMEM((FAST_CHUNKS, CHUNK), jnp.int32),
            pltpu.VMEM((FAST_CHUNKS * CHUNK, D_MODEL), jnp.float32),
            [pltpu.SemaphoreType.DMA] * FAST_CHUNKS,
            pltpu.SemaphoreType.DMA,
        ],
    )
    def emb_kernel(x_hbm, lut_hbm, out_hbm, idx_v, rows_v, gsems, wsem):
        c = lax.axis_index("c")
        s = lax.axis_index("s")
        first = jnp.where(c == 1, s * FAST_CHUNKS,
                          NUM_SUBCORES * FAST_CHUNKS + s * SLOW_CHUNKS)

        def chunk_ops(j):
            cid = first + j
            row = cid // (b1 // CHUNK)
            col = (cid % (b1 // CHUNK)) * CHUNK
            idx_load = pltpu.make_async_copy(
                x_hbm.at[row, pl.ds(col, CHUNK)], idx_v.at[j], gsems[j])
            gather = pltpu.make_async_copy(
                lut_hbm.at[idx_v.at[j]],
                rows_v.at[pl.ds(j * CHUNK, CHUNK)], gsems[j])
            write = pltpu.make_async_copy(
                rows_v.at[pl.ds(j * CHUNK, CHUNK)],
                out_hbm.at[pl.ds(cid * CHUNK, CHUNK)], wsem)
            return idx_load, gather, write

        def scale_rows(off):
            @plsc.parallel_loop(off, off + CHUNK, unroll=1)
            def _(r):
                for c8 in range(D_MODEL // LANES):
                    sl = rows_v[r, pl.ds(c8 * LANES, LANES)]
                    rows_v[r, pl.ds(c8 * LANES, LANES)] = sl * SCALE

        for j in range(SLOW_CHUNKS):
            chunk_ops(j)[0].start()

        for j in range(SLOW_CHUNKS):
            idx_load, gather, _ = chunk_ops(j)
            idx_load.wait()
            gather.start()

        @pl.loop(0, jnp.where(c == 1, 1, 0))
        def _(t):
            extras = list(range(SLOW_CHUNKS, FAST_CHUNKS))
            for j in extras:
                chunk_ops(j)[0].start()
            for j in extras:
                idx_load, gather, _ = chunk_ops(j)
                idx_load.wait()
                gather.start()
            for j in extras:
                _, gather, write = chunk_ops(j)
                gather.wait()
                scale_rows(j * CHUNK)
                write.start()
            for j in extras:
                chunk_ops(j)[2].wait()

        for j in range(SLOW_CHUNKS):
            _, gather, write = chunk_ops(j)
            gather.wait()
            scale_rows(j * CHUNK)
            write.start()

        for j in range(SLOW_CHUNKS):
            chunk_ops(j)[2].wait()

    return emb_kernel


def kernel(x, lut):
    b0, b1 = x.shape
    if x.dtype != jnp.int32:
        x = x.astype(jnp.int32)
    out = _build(b0, b1)(x, lut)
    return out.reshape(b0, b1, D_MODEL)
